# Initial kernel scaffold; baseline (speedup 1.0000x reference)
import jax
import jax.numpy as jnp
from jax.experimental import pallas as pl


def kernel(sampled_state, scatter_index, moment_slices, carry):
    raise NotImplementedError("write your pallas kernel here")


# speedup vs baseline: 15.8103x; 15.8103x over previous
import dataclasses
import functools

import jax
import jax.numpy as jnp
from jax import lax
from jax.experimental import pallas as pl
from jax.experimental.pallas import tpu as pltpu
from jax.experimental.pallas import tpu_sc as plsc

FLAT = 1048576
GROUPS = 4194304
NC = 2
NS = 16
NW = NC * NS
L = 16

SCAT_PER_TILE = FLAT // NS
SCAT_CHUNK = 8192
GRP_PER_TILE = GROUPS // NW
GCHUNK = 4096


def kernel(sampled_state, scatter_index, moment_slices, carry):
    ms_flat = moment_slices.reshape(-1)
    mesh = plsc.VectorSubcoreMesh(core_axis_name="c", subcore_axis_name="s")
    cp = pltpu.CompilerParams()
    if "needs_layout_passes" in pltpu.CompilerParams.__dataclass_fields__:
        cp = dataclasses.replace(cp, needs_layout_passes=False)

    @functools.partial(
        pl.kernel,
        compiler_params=cp,
        out_type=jax.ShapeDtypeStruct((GROUPS,), jnp.float32),
        mesh=mesh,
        scratch_types=[
            pltpu.VMEM_SHARED((FLAT,), jnp.float32),
            pltpu.VMEM((SCAT_CHUNK,), jnp.float32),
            pltpu.VMEM((SCAT_CHUNK,), jnp.int32),
            pltpu.VMEM((2 * GCHUNK,), jnp.int32),
            pltpu.VMEM((2 * GCHUNK,), jnp.float32),
            pltpu.VMEM((GCHUNK,), jnp.float32),
            pltpu.VMEM((GCHUNK,), jnp.float32),
        ],
    )
    def k(samp_hbm, sidx_hbm, ms_hbm, carry_hbm, out_hbm,
          table, sv, si, mi, gv, cv, ov):
        c = lax.axis_index("c")
        s = lax.axis_index("s")
        wid = s * NC + c

        tile_base = s * SCAT_PER_TILE

        @pl.loop(0, SCAT_PER_TILE, step=SCAT_CHUNK)
        def _(off):
            base = tile_base + off
            pltpu.sync_copy(samp_hbm.at[pl.ds(base, SCAT_CHUNK)], sv)
            pltpu.sync_copy(sidx_hbm.at[pl.ds(base, SCAT_CHUNK)], si)
            pltpu.sync_copy(sv, table.at[si])

        plsc.subcore_barrier()

        grp_base = wid * GRP_PER_TILE
        iota2 = lax.iota(jnp.int32, L) * 2

        @pl.loop(0, GRP_PER_TILE, step=GCHUNK)
        def _(goff):
            g0 = grp_base + goff
            pltpu.sync_copy(ms_hbm.at[pl.ds(2 * g0, 2 * GCHUNK)], mi)
            pltpu.sync_copy(table.at[mi], gv)
            pltpu.sync_copy(carry_hbm.at[pl.ds(g0, GCHUNK)], cv)

            @pl.loop(0, GCHUNK, step=L)
            def _(v):
                v = pl.multiple_of(v, L)
                eidx = iota2 + 2 * v
                a = plsc.load_gather(gv, [eidx])
                b = plsc.load_gather(gv, [eidx + 1])
                ov[pl.ds(v, L)] = a * b + cv[pl.ds(v, L)]

            pltpu.sync_copy(ov, out_hbm.at[pl.ds(g0, GCHUNK)])

    return k(sampled_state, scatter_index, ms_flat, carry)


# speedup vs baseline: 318.2598x; 20.1299x over previous
import dataclasses
import functools

import jax
import jax.numpy as jnp
from jax import lax
from jax.experimental import pallas as pl
from jax.experimental.pallas import tpu as pltpu
from jax.experimental.pallas import tpu_sc as plsc

FLAT = 1048576
GROUPS = 4194304
NC = 2
NS = 16
NW = NC * NS
L = 16

SCAT_PER_TILE = FLAT // NS
SCAT_CHUNK = 8192
GRP_PER_TILE = GROUPS // NW
GCHUNK = 4096


def kernel(sampled_state, scatter_index, moment_slices, carry):
    m0 = moment_slices[:, 0]
    m1 = moment_slices[:, 1]
    mesh = plsc.VectorSubcoreMesh(core_axis_name="c", subcore_axis_name="s")
    cp = pltpu.CompilerParams()
    if "needs_layout_passes" in pltpu.CompilerParams.__dataclass_fields__:
        cp = dataclasses.replace(cp, needs_layout_passes=False)

    @functools.partial(
        pl.kernel,
        compiler_params=cp,
        out_type=jax.ShapeDtypeStruct((GROUPS,), jnp.float32),
        mesh=mesh,
        scratch_types=[
            pltpu.VMEM_SHARED((FLAT,), jnp.float32),
            pltpu.VMEM((SCAT_CHUNK,), jnp.float32),
            pltpu.VMEM((SCAT_CHUNK,), jnp.int32),
            pltpu.VMEM((GCHUNK,), jnp.int32),
            pltpu.VMEM((GCHUNK,), jnp.int32),
            pltpu.VMEM((GCHUNK,), jnp.float32),
            pltpu.VMEM((GCHUNK,), jnp.float32),
            pltpu.VMEM((GCHUNK,), jnp.float32),
            pltpu.VMEM((GCHUNK,), jnp.float32),
        ],
    )
    def k(samp_hbm, sidx_hbm, m0_hbm, m1_hbm, carry_hbm, out_hbm,
          table, sv, si, i0, i1, v0, v1, cv, ov):
        c = lax.axis_index("c")
        s = lax.axis_index("s")
        wid = s * NC + c

        tile_base = s * SCAT_PER_TILE

        @pl.loop(0, SCAT_PER_TILE, step=SCAT_CHUNK)
        def _(off):
            base = tile_base + off
            pltpu.sync_copy(samp_hbm.at[pl.ds(base, SCAT_CHUNK)], sv)
            pltpu.sync_copy(sidx_hbm.at[pl.ds(base, SCAT_CHUNK)], si)
            pltpu.sync_copy(sv, table.at[si])

        plsc.subcore_barrier()

        grp_base = wid * GRP_PER_TILE

        @pl.loop(0, GRP_PER_TILE, step=GCHUNK)
        def _(goff):
            g0 = grp_base + goff
            pltpu.sync_copy(m0_hbm.at[pl.ds(g0, GCHUNK)], i0)
            pltpu.sync_copy(m1_hbm.at[pl.ds(g0, GCHUNK)], i1)
            pltpu.sync_copy(table.at[i0], v0)
            pltpu.sync_copy(table.at[i1], v1)
            pltpu.sync_copy(carry_hbm.at[pl.ds(g0, GCHUNK)], cv)

            @pl.loop(0, GCHUNK, step=L)
            def _(v):
                v = pl.multiple_of(v, L)
                sl = pl.ds(v, L)
                ov[sl] = v0[sl] * v1[sl] + cv[sl]

            pltpu.sync_copy(ov, out_hbm.at[pl.ds(g0, GCHUNK)])

    return k(sampled_state, scatter_index, m0, m1, carry)


# speedup vs baseline: 555.3141x; 1.7448x over previous
import dataclasses
import functools

import jax
import jax.numpy as jnp
from jax import lax
from jax.experimental import pallas as pl
from jax.experimental.pallas import tpu as pltpu
from jax.experimental.pallas import tpu_sc as plsc

FLAT = 1048576
GROUPS = 4194304
NC = 2
NS = 16
NW = NC * NS
L = 16

SCAT_PER_TILE = FLAT // NS
SCAT_CHUNK = 2048
NSCH = SCAT_PER_TILE // SCAT_CHUNK
GRP_PER_TILE = GROUPS // NW
GCHUNK = 4096
NCH = GRP_PER_TILE // GCHUNK


def kernel(sampled_state, scatter_index, moment_slices, carry):
    m0 = moment_slices[:, 0]
    m1 = moment_slices[:, 1]
    mesh = plsc.VectorSubcoreMesh(core_axis_name="c", subcore_axis_name="s")
    cp = pltpu.CompilerParams()
    if "needs_layout_passes" in pltpu.CompilerParams.__dataclass_fields__:
        cp = dataclasses.replace(cp, needs_layout_passes=False)

    f32 = jnp.float32
    i32 = jnp.int32

    @functools.partial(
        pl.kernel,
        compiler_params=cp,
        out_type=jax.ShapeDtypeStruct((GROUPS,), f32),
        mesh=mesh,
        scratch_types=[
            pltpu.VMEM_SHARED((FLAT,), f32),
            pltpu.VMEM((SCAT_CHUNK,), f32),
            pltpu.VMEM((SCAT_CHUNK,), f32),
            pltpu.VMEM((SCAT_CHUNK,), i32),
            pltpu.VMEM((SCAT_CHUNK,), i32),
            pltpu.VMEM((GCHUNK,), i32),
            pltpu.VMEM((GCHUNK,), i32),
            pltpu.VMEM((GCHUNK,), i32),
            pltpu.VMEM((GCHUNK,), i32),
            pltpu.VMEM((GCHUNK,), f32),
            pltpu.VMEM((GCHUNK,), f32),
            pltpu.VMEM((GCHUNK,), f32),
            pltpu.VMEM((GCHUNK,), f32),
            pltpu.VMEM((GCHUNK,), f32),
            pltpu.VMEM((GCHUNK,), f32),
            pltpu.VMEM((GCHUNK,), f32),
            pltpu.VMEM((GCHUNK,), f32),
            pltpu.SemaphoreType.DMA((2,)),
            pltpu.SemaphoreType.DMA((2,)),
            pltpu.SemaphoreType.DMA((2,)),
            pltpu.SemaphoreType.DMA((2,)),
            pltpu.SemaphoreType.DMA((2,)),
        ],
    )
    def k(samp_hbm, sidx_hbm, m0_hbm, m1_hbm, carry_hbm, out_hbm,
          table, sva, svb, sia, sib, i0a, i0b, i1a, i1b,
          v0a, v0b, v1a, v1b, cva, cvb, ova, ovb,
          s_idx, s_gat, s_out, s_p1i, s_p1s):
        c = lax.axis_index("c")
        s = lax.axis_index("s")
        wid = s * NC + c

        sv = (sva, svb)
        si = (sia, sib)
        i0 = (i0a, i0b)
        i1 = (i1a, i1b)
        v0 = (v0a, v0b)
        v1 = (v1a, v1b)
        cv = (cva, cvb)
        ov = (ova, ovb)

        tile_base = s * SCAT_PER_TILE

        def p1_start_in(kk, b):
            base = tile_base + kk * SCAT_CHUNK
            pltpu.async_copy(samp_hbm.at[pl.ds(base, SCAT_CHUNK)],
                             sv[b], s_p1i.at[b])
            pltpu.async_copy(sidx_hbm.at[pl.ds(base, SCAT_CHUNK)],
                             si[b], s_p1i.at[b])

        def p1_wait_in(b):
            pltpu.make_async_copy(samp_hbm.at[pl.ds(0, SCAT_CHUNK)],
                                  sv[b], s_p1i.at[b]).wait()
            pltpu.make_async_copy(sidx_hbm.at[pl.ds(0, SCAT_CHUNK)],
                                  si[b], s_p1i.at[b]).wait()

        def p1_scatter(b):
            pltpu.async_copy(sv[b], table.at[si[b]], s_p1s.at[b])

        def p1_wait_scatter(b):
            pltpu.make_async_copy(sv[b], table.at[si[b]],
                                  s_p1s.at[b]).wait()

        p1_start_in(0, 0)
        p1_start_in(1, 1)

        @pl.loop(0, NSCH, step=2)
        def _(kk):
            p1_wait_in(0)
            p1_scatter(0)
            p1_wait_scatter(0)

            @pl.when(kk + 2 < NSCH)
            def _():
                p1_start_in(kk + 2, 0)

            p1_wait_in(1)
            p1_scatter(1)
            p1_wait_scatter(1)

            @pl.when(kk + 3 < NSCH)
            def _():
                p1_start_in(kk + 3, 1)

        plsc.subcore_barrier()

        grp_base = wid * GRP_PER_TILE

        def start_idx(kk, b):
            g0 = grp_base + kk * GCHUNK
            pltpu.async_copy(m0_hbm.at[pl.ds(g0, GCHUNK)], i0[b], s_idx.at[b])
            pltpu.async_copy(m1_hbm.at[pl.ds(g0, GCHUNK)], i1[b], s_idx.at[b])
            pltpu.async_copy(carry_hbm.at[pl.ds(g0, GCHUNK)], cv[b],
                             s_idx.at[b])

        def wait_idx(b):
            pltpu.make_async_copy(m0_hbm.at[pl.ds(0, GCHUNK)], i0[b],
                                  s_idx.at[b]).wait()
            pltpu.make_async_copy(m1_hbm.at[pl.ds(0, GCHUNK)], i1[b],
                                  s_idx.at[b]).wait()
            pltpu.make_async_copy(carry_hbm.at[pl.ds(0, GCHUNK)], cv[b],
                                  s_idx.at[b]).wait()

        def start_gather(b):
            pltpu.async_copy(table.at[i0[b]], v0[b], s_gat.at[b])
            pltpu.async_copy(table.at[i1[b]], v1[b], s_gat.at[b])

        def wait_gather(b):
            pltpu.make_async_copy(table.at[i0[b]], v0[b], s_gat.at[b]).wait()
            pltpu.make_async_copy(table.at[i1[b]], v1[b], s_gat.at[b]).wait()

        def start_out(kk, b):
            g0 = grp_base + kk * GCHUNK
            pltpu.async_copy(ov[b], out_hbm.at[pl.ds(g0, GCHUNK)],
                             s_out.at[b])

        def wait_out(b):
            pltpu.make_async_copy(ov[b], out_hbm.at[pl.ds(0, GCHUNK)],
                                  s_out.at[b]).wait()

        def compute(b):
            @pl.loop(0, GCHUNK, step=L)
            def _(v):
                v = pl.multiple_of(v, L)
                sl = pl.ds(v, L)
                ov[b][sl] = v0[b][sl] * v1[b][sl] + cv[b][sl]

        start_idx(0, 0)
        wait_idx(0)
        start_gather(0)
        start_idx(1, 1)

        @pl.loop(0, NCH, step=2)
        def _(kk):
            wait_gather(0)
            wait_idx(1)
            start_gather(1)

            @pl.when(kk >= 2)
            def _():
                wait_out(0)
            compute(0)
            start_out(kk, 0)

            @pl.when(kk + 2 < NCH)
            def _():
                start_idx(kk + 2, 0)

            wait_gather(1)

            @pl.when(kk + 2 < NCH)
            def _():
                wait_idx(0)
                start_gather(0)

            @pl.when(kk >= 2)
            def _():
                wait_out(1)
            compute(1)
            start_out(kk + 1, 1)

            @pl.when(kk + 3 < NCH)
            def _():
                start_idx(kk + 3, 1)

        wait_out(0)
        wait_out(1)

    return k(sampled_state, scatter_index, m0, m1, carry)
